# Initial kernel scaffold; baseline (speedup 1.0000x reference)
import jax
import jax.numpy as jnp
from jax.experimental import pallas as pl


def kernel(points, colors):
    raise NotImplementedError("write your pallas kernel here")


# speedup vs baseline: 9.9467x; 9.9467x over previous
import functools

import jax
import jax.numpy as jnp
from jax import lax
from jax.experimental import pallas as pl
from jax.experimental.pallas import tpu as pltpu
from jax.experimental.pallas import tpu_sc as plsc

H, W = 480, 640
P = H * W
FX, FY, CX, CY = 500.0, 500.0, 320.0, 240.0
NPTS = 2_000_000
PN = 2 ** 21
PAD = PN - NPTS
SENT = 2147483647

NSTRIP = 8
STRIP = P // NSTRIP
NCHUNK = 4
CPTS = PN // NCHUNK
CH = 4096
TS = 960
QUARTER = STRIP // NCHUNK

PROWS = PN // 128
ORERows = P // 128


def _prep_body(x_r, y_r, z_r, r_r, g_r, b_r, lin_r, key_r, c30_r):
    z = z_r[...] + 3.0
    u = jnp.round(FX * x_r[...] / z + CX).astype(jnp.int32)
    v = jnp.round(FY * y_r[...] / z + CY).astype(jnp.int32)
    valid = (z > 1e-6) & (u >= 0) & (u < W) & (v >= 0) & (v < H)
    lin_r[...] = jnp.where(valid, v * W + u, SENT)
    zb = lax.bitcast_convert_type(z, jnp.int32)
    key_r[...] = jnp.where(valid, zb, SENT)

    def q(c_ref):
        return jnp.clip((c_ref[...] * 1024.0).astype(jnp.int32), 0, 1023)

    c30_r[...] = (q(r_r) << 20) | (q(g_r) << 10) | q(b_r)


_sc_mesh = plsc.VectorSubcoreMesh(core_axis_name="c", subcore_axis_name="s")


@functools.partial(
    pl.kernel,
    out_type=jax.ShapeDtypeStruct((P,), jnp.int32),
    mesh=_sc_mesh,
    compiler_params=pltpu.CompilerParams(needs_layout_passes=False),
    scratch_types=[
        pltpu.VMEM((STRIP,), jnp.int32),
        pltpu.VMEM((STRIP,), jnp.int32),
        pltpu.VMEM((CH,), jnp.int32),
        pltpu.VMEM((CH,), jnp.int32),
        pltpu.VMEM((CH,), jnp.int32),
        pltpu.VMEM_SHARED((16 * STRIP,), jnp.int32),
    ],
)
def _sc_scatter(lin_hbm, key_hbm, c30_hbm, out_hbm,
                d_tile, c_tile, lin_b, key_b, c_b, shared):
    cid = lax.axis_index("c")
    sid = lax.axis_index("s")
    strip = 4 * cid + sid // 4
    chunk = sid % 4
    strip_lo = strip * STRIP
    base = chunk * CPTS
    srow = (sid // 4) * 4

    maxv = jnp.full((16,), SENT, jnp.int32)
    negv = jnp.full((16,), -1, jnp.int32)

    def init_body(i, carry):
        d_tile[pl.ds(i * 16, 16)] = maxv
        c_tile[pl.ds(i * 16, 16)] = negv
        return carry

    lax.fori_loop(0, STRIP // 16, init_body, 0)

    def blk_a(b, carry):
        off = pl.multiple_of(base + b * CH, CH)
        pltpu.sync_copy(lin_hbm.at[pl.ds(off, CH)], lin_b)
        pltpu.sync_copy(key_hbm.at[pl.ds(off, CH)], key_b)

        def vec_a(i, c2):
            l = lin_b[pl.ds(i * 16, 16)]
            k = key_b[pl.ds(i * 16, 16)]
            li = l - strip_lo
            m = (li >= 0) & (li < STRIP)
            lic = jnp.where(m, li, 0)
            d = plsc.load_gather(d_tile, [lic])
            need = m & (k < d)

            def w_cond(mask):
                return jnp.sum(mask.astype(jnp.int32)) > 0

            def w_body(mask):
                plsc.store_scatter(d_tile, [lic], k, mask=mask)
                d2 = plsc.load_gather(d_tile, [lic])
                return mask & (k < d2)

            lax.while_loop(w_cond, w_body, need)
            return c2

        lax.fori_loop(0, CH // 16, vec_a, 0)
        return carry

    lax.fori_loop(0, CPTS // CH, blk_a, 0)

    my_slot = pl.multiple_of(sid * STRIP, STRIP)
    pltpu.sync_copy(d_tile, shared.at[pl.ds(my_slot, STRIP)])
    plsc.subcore_barrier()

    def merge_d(t, carry):
        for j in range(4):
            soff = pl.multiple_of((srow + j) * STRIP + t * TS, TS)
            pltpu.sync_copy(shared.at[pl.ds(soff, TS)],
                            lin_b.at[pl.ds(j * TS, TS)])

        def mv(i, c2):
            a = lin_b[pl.ds(0 * TS + i * 16, 16)]
            b2 = lin_b[pl.ds(1 * TS + i * 16, 16)]
            c3 = lin_b[pl.ds(2 * TS + i * 16, 16)]
            e = lin_b[pl.ds(3 * TS + i * 16, 16)]
            d_tile[pl.ds(t * TS + i * 16, 16)] = jnp.minimum(
                jnp.minimum(a, b2), jnp.minimum(c3, e))
            return c2

        lax.fori_loop(0, TS // 16, mv, 0)
        return carry

    lax.fori_loop(0, STRIP // TS, merge_d, 0)
    plsc.subcore_barrier()

    def blk_b(b, carry):
        off = pl.multiple_of(base + b * CH, CH)
        pltpu.sync_copy(lin_hbm.at[pl.ds(off, CH)], lin_b)
        pltpu.sync_copy(key_hbm.at[pl.ds(off, CH)], key_b)
        pltpu.sync_copy(c30_hbm.at[pl.ds(off, CH)], c_b)

        def vec_b(i, c2):
            l = lin_b[pl.ds(i * 16, 16)]
            k = key_b[pl.ds(i * 16, 16)]
            c = c_b[pl.ds(i * 16, 16)]
            li = l - strip_lo
            m = (li >= 0) & (li < STRIP)
            lic = jnp.where(m, li, 0)
            d = plsc.load_gather(d_tile, [lic])
            win = m & (k == d)
            plsc.store_scatter(c_tile, [lic], c, mask=win)
            return c2

        lax.fori_loop(0, CH // 16, vec_b, 0)
        return carry

    lax.fori_loop(0, CPTS // CH, blk_b, 0)

    my_slot2 = pl.multiple_of(sid * STRIP, STRIP)
    pltpu.sync_copy(c_tile, shared.at[pl.ds(my_slot2, STRIP)])
    plsc.subcore_barrier()
    qo = chunk * QUARTER

    def merge_c(t, carry):
        for j in range(4):
            soff = pl.multiple_of((srow + j) * STRIP + qo + t * TS, TS)
            pltpu.sync_copy(shared.at[pl.ds(soff, TS)],
                            lin_b.at[pl.ds(j * TS, TS)])

        def cv(i, c2):
            a = lin_b[pl.ds(0 * TS + i * 16, 16)]
            b2 = lin_b[pl.ds(1 * TS + i * 16, 16)]
            c3 = lin_b[pl.ds(2 * TS + i * 16, 16)]
            e = lin_b[pl.ds(3 * TS + i * 16, 16)]
            key_b[pl.ds(i * 16, 16)] = jnp.maximum(
                jnp.maximum(a, b2), jnp.maximum(c3, e))
            return c2

        lax.fori_loop(0, TS // 16, cv, 0)
        ooff = pl.multiple_of(strip_lo + qo + t * TS, TS)
        pltpu.sync_copy(key_b.at[pl.ds(0, TS)],
                        out_hbm.at[pl.ds(ooff, TS)])
        return carry

    lax.fori_loop(0, QUARTER // TS, merge_c, 0)


def _unpack_body(c_r, r_r, g_r, b_r):
    c = c_r[...]
    cov = c >= 0

    def ch(shift):
        q = (c >> shift) & 1023
        return jnp.where(cov, (q.astype(jnp.float32) + 0.5) * (1.0 / 1024.0),
                         0.0)

    r_r[...] = ch(20)
    g_r[...] = ch(10)
    b_r[...] = ch(0)


def kernel(points, colors):
    xs = jnp.pad(points[:, 0], (0, PAD)).reshape(PROWS, 128)
    ys = jnp.pad(points[:, 1], (0, PAD)).reshape(PROWS, 128)
    zs = jnp.pad(points[:, 2], (0, PAD),
                 constant_values=-3.0).reshape(PROWS, 128)
    rs = jnp.pad(colors[:, 0], (0, PAD)).reshape(PROWS, 128)
    gs = jnp.pad(colors[:, 1], (0, PAD)).reshape(PROWS, 128)
    bs = jnp.pad(colors[:, 2], (0, PAD)).reshape(PROWS, 128)
    lin, key, c30 = pl.pallas_call(
        _prep_body,
        grid=(16,),
        in_specs=[pl.BlockSpec((1024, 128), lambda i: (i, 0))] * 6,
        out_specs=[pl.BlockSpec((1024, 128), lambda i: (i, 0))] * 3,
        out_shape=[jax.ShapeDtypeStruct((PROWS, 128), jnp.int32)] * 3,
    )(xs, ys, zs, rs, gs, bs)
    c30m = _sc_scatter(lin.reshape(PN), key.reshape(PN), c30.reshape(PN))
    cr, cg, cb = pl.pallas_call(
        _unpack_body,
        out_shape=[jax.ShapeDtypeStruct((ORERows, 128), jnp.float32)] * 3,
    )(c30m.reshape(ORERows, 128))
    return jnp.stack([cr, cg, cb], axis=-1).reshape(H, W, 3)


# speedup vs baseline: 13.9368x; 1.4011x over previous
import functools

import jax
import jax.numpy as jnp
from jax import lax
from jax.experimental import pallas as pl
from jax.experimental.pallas import tpu as pltpu
from jax.experimental.pallas import tpu_sc as plsc

H, W = 480, 640
P = H * W
FX, FY, CX, CY = 500.0, 500.0, 320.0, 240.0
NPTS = 2_000_000
PN = 2 ** 21
PAD = PN - NPTS
SENT = 2147483647

NSTRIP = 8
STRIP = P // NSTRIP
NCHUNK = 4
CPTS = PN // NCHUNK
CH = 4096
TS = 960
QUARTER = STRIP // NCHUNK

PROWS = PN // 128
OROWS = P // 128


def _prep_body(x_r, y_r, z_r, r_r, g_r, b_r, lin_r, key_r, c30_r):
    z = z_r[...] + 3.0
    u = jnp.round(FX * x_r[...] / z + CX).astype(jnp.int32)
    v = jnp.round(FY * y_r[...] / z + CY).astype(jnp.int32)
    valid = (z > 1e-6) & (u >= 0) & (u < W) & (v >= 0) & (v < H)
    lin_r[...] = jnp.where(valid, v * W + u, SENT)
    zb = lax.bitcast_convert_type(z, jnp.int32)
    key_r[...] = jnp.where(valid, zb, SENT)

    def q(c_ref):
        return jnp.clip((c_ref[...] * 1024.0).astype(jnp.int32), 0, 1023)

    c30_r[...] = (q(r_r) << 20) | (q(g_r) << 10) | q(b_r)


_sc_mesh = plsc.VectorSubcoreMesh(core_axis_name="c", subcore_axis_name="s")


@functools.partial(
    pl.kernel,
    out_type=(
        jax.ShapeDtypeStruct((P,), jnp.int32),
        jax.ShapeDtypeStruct((32 * STRIP,), jnp.int32),
        jax.ShapeDtypeStruct((32 * STRIP,), jnp.int32),
    ),
    mesh=_sc_mesh,
    compiler_params=pltpu.CompilerParams(needs_layout_passes=False),
    scratch_types=[
        pltpu.VMEM((STRIP,), jnp.int32),
        pltpu.VMEM((STRIP,), jnp.int32),
        pltpu.VMEM((CH,), jnp.int32),
        pltpu.VMEM((CH,), jnp.int32),
        pltpu.VMEM((CH,), jnp.int32),
    ],
)
def _sc_scatter(lin_hbm, key_hbm, c30_hbm, out_hbm, dpart_hbm, cpart_hbm,
                d_tile, c_tile, lin_b, key_b, c_b):
    cid = lax.axis_index("c")
    sid = lax.axis_index("s")
    strip = 4 * cid + sid // 4
    chunk = sid % 4
    strip_lo = strip * STRIP
    base = chunk * CPTS
    wid = cid * 16 + sid
    base_row = cid * 16 + (sid // 4) * 4

    maxv = jnp.full((16,), SENT, jnp.int32)
    negv = jnp.full((16,), -1, jnp.int32)

    def init_body(i, carry):
        d_tile[pl.ds(i * 16, 16)] = maxv
        c_tile[pl.ds(i * 16, 16)] = negv
        return carry

    lax.fori_loop(0, STRIP // 16, init_body, 0)

    def blk(b, carry):
        off = pl.multiple_of(base + b * CH, CH)
        pltpu.sync_copy(lin_hbm.at[pl.ds(off, CH)], lin_b)
        pltpu.sync_copy(key_hbm.at[pl.ds(off, CH)], key_b)
        pltpu.sync_copy(c30_hbm.at[pl.ds(off, CH)], c_b)

        def vec(i, c2):
            l = lin_b[pl.ds(i * 16, 16)]
            k = key_b[pl.ds(i * 16, 16)]
            c = c_b[pl.ds(i * 16, 16)]
            li = l - strip_lo
            m = (li >= 0) & (li < STRIP)
            lic = jnp.where(m, li, 0)
            d = plsc.load_gather(d_tile, [lic])
            need = m & (k < d)
            cnt = plsc.all_reduce_population_count(need)[0]

            @pl.when(cnt > 0)
            def _():
                def step(mask):
                    plsc.store_scatter(d_tile, [lic], k, mask=mask)
                    d2 = plsc.load_gather(d_tile, [lic])
                    won = mask & (k == d2)
                    plsc.store_scatter(c_tile, [lic], c, mask=won)
                    return mask & (k < d2)

                def w_cond(mask):
                    return plsc.all_reduce_population_count(mask)[0] > 0

                lax.while_loop(w_cond, step, step(need))

            return c2

        lax.fori_loop(0, CH // 16, vec, 0)
        return carry

    lax.fori_loop(0, CPTS // CH, blk, 0)

    poff = pl.multiple_of(wid * STRIP, STRIP)
    pltpu.sync_copy(d_tile, dpart_hbm.at[pl.ds(poff, STRIP)])
    pltpu.sync_copy(c_tile, cpart_hbm.at[pl.ds(poff, STRIP)])
    plsc.subcore_barrier()

    qo = chunk * QUARTER
    for t in range(QUARTER // TS):
        for j in range(4):
            doff = pl.multiple_of((base_row + j) * STRIP + qo + t * TS, TS)
            pltpu.sync_copy(dpart_hbm.at[pl.ds(doff, TS)],
                            lin_b.at[pl.ds(j * TS, TS)])
            pltpu.sync_copy(cpart_hbm.at[pl.ds(doff, TS)],
                            key_b.at[pl.ds(j * TS, TS)])

        def mv(i, c2, _t=t):
            d0 = lin_b[pl.ds(0 * TS + i * 16, 16)]
            d1 = lin_b[pl.ds(1 * TS + i * 16, 16)]
            d2 = lin_b[pl.ds(2 * TS + i * 16, 16)]
            d3 = lin_b[pl.ds(3 * TS + i * 16, 16)]
            c0 = key_b[pl.ds(0 * TS + i * 16, 16)]
            c1 = key_b[pl.ds(1 * TS + i * 16, 16)]
            c2_ = key_b[pl.ds(2 * TS + i * 16, 16)]
            c3 = key_b[pl.ds(3 * TS + i * 16, 16)]
            d01 = jnp.minimum(d0, d1)
            c01 = jnp.where(d0 <= d1, c0, c1)
            d23 = jnp.minimum(d2, d3)
            c23 = jnp.where(d2 <= d3, c2_, c3)
            c_b[pl.ds(i * 16, 16)] = jnp.where(d01 <= d23, c01, c23)
            return c2

        lax.fori_loop(0, TS // 16, mv, 0)
        ooff = pl.multiple_of(strip_lo + qo + t * TS, TS)
        pltpu.sync_copy(c_b.at[pl.ds(0, TS)], out_hbm.at[pl.ds(ooff, TS)])


def _unpack_body(c_r, r_r, g_r, b_r):
    c = c_r[...]
    cov = c >= 0

    def ch(shift):
        q = (c >> shift) & 1023
        return jnp.where(cov, (q.astype(jnp.float32) + 0.5) * (1.0 / 1024.0),
                         0.0)

    r_r[...] = ch(20)
    g_r[...] = ch(10)
    b_r[...] = ch(0)


def kernel(points, colors):
    xs = jnp.pad(points[:, 0], (0, PAD)).reshape(PROWS, 128)
    ys = jnp.pad(points[:, 1], (0, PAD)).reshape(PROWS, 128)
    zs = jnp.pad(points[:, 2], (0, PAD),
                 constant_values=-3.0).reshape(PROWS, 128)
    rs = jnp.pad(colors[:, 0], (0, PAD)).reshape(PROWS, 128)
    gs = jnp.pad(colors[:, 1], (0, PAD)).reshape(PROWS, 128)
    bs = jnp.pad(colors[:, 2], (0, PAD)).reshape(PROWS, 128)
    lin, key, c30 = pl.pallas_call(
        _prep_body,
        grid=(16,),
        in_specs=[pl.BlockSpec((1024, 128), lambda i: (i, 0))] * 6,
        out_specs=[pl.BlockSpec((1024, 128), lambda i: (i, 0))] * 3,
        out_shape=[jax.ShapeDtypeStruct((PROWS, 128), jnp.int32)] * 3,
    )(xs, ys, zs, rs, gs, bs)
    c30m, _, _ = _sc_scatter(lin.reshape(PN), key.reshape(PN),
                             c30.reshape(PN))
    cr, cg, cb = pl.pallas_call(
        _unpack_body,
        out_shape=[jax.ShapeDtypeStruct((OROWS, 128), jnp.float32)] * 3,
    )(c30m.reshape(OROWS, 128))
    return jnp.stack([cr, cg, cb], axis=-1).reshape(H, W, 3)


# speedup vs baseline: 15.6786x; 1.1250x over previous
import functools

import jax
import jax.numpy as jnp
from jax import lax
from jax.experimental import pallas as pl
from jax.experimental.pallas import tpu as pltpu
from jax.experimental.pallas import tpu_sc as plsc

H, W = 480, 640
P = H * W
FX, FY, CX, CY = 500.0, 500.0, 320.0, 240.0
NPTS = 2_000_000
PN = 2 ** 21
PAD = PN - NPTS
SENT = 2147483647

NSTRIP = 8
STRIP = P // NSTRIP
NCHUNK = 4
CPTS = PN // NCHUNK
CH = 16384
TS = 960
QUARTER = STRIP // NCHUNK

PROWS = PN // 128
OROWS = P // 128


def _prep_body(x_r, y_r, z_r, r_r, g_r, b_r, lin_r, key_r, c30_r):
    z = z_r[...] + 3.0
    u = jnp.round(FX * x_r[...] / z + CX).astype(jnp.int32)
    v = jnp.round(FY * y_r[...] / z + CY).astype(jnp.int32)
    valid = (z > 1e-6) & (u >= 0) & (u < W) & (v >= 0) & (v < H)
    lin_r[...] = jnp.where(valid, v * W + u, SENT)
    zb = lax.bitcast_convert_type(z, jnp.int32)
    key_r[...] = jnp.where(valid, zb, SENT)

    def q(c_ref):
        return jnp.clip((c_ref[...] * 1024.0).astype(jnp.int32), 0, 1023)

    c30_r[...] = (q(r_r) << 20) | (q(g_r) << 10) | q(b_r)


_sc_mesh = plsc.VectorSubcoreMesh(core_axis_name="c", subcore_axis_name="s")


@functools.partial(
    pl.kernel,
    out_type=(
        jax.ShapeDtypeStruct((P,), jnp.int32),
        jax.ShapeDtypeStruct((32 * STRIP,), jnp.int32),
        jax.ShapeDtypeStruct((32 * STRIP,), jnp.int32),
    ),
    mesh=_sc_mesh,
    compiler_params=pltpu.CompilerParams(needs_layout_passes=False),
    scratch_types=[
        pltpu.VMEM((STRIP,), jnp.int32),
        pltpu.VMEM((STRIP,), jnp.int32),
        pltpu.VMEM((CH,), jnp.int32),
        pltpu.VMEM((CH,), jnp.int32),
        pltpu.VMEM((CH,), jnp.int32),
    ],
)
def _sc_scatter(lin_hbm, key_hbm, c30_hbm, out_hbm, dpart_hbm, cpart_hbm,
                d_tile, c_tile, lin_b, key_b, c_b):
    cid = lax.axis_index("c")
    sid = lax.axis_index("s")
    strip = 4 * cid + sid // 4
    chunk = sid % 4
    strip_lo = strip * STRIP
    base = chunk * CPTS
    wid = cid * 16 + sid
    base_row = cid * 16 + (sid // 4) * 4

    maxv = jnp.full((16,), SENT, jnp.int32)
    negv = jnp.full((16,), -1, jnp.int32)

    def init_body(i, carry):
        d_tile[pl.ds(i * 16, 16)] = maxv
        c_tile[pl.ds(i * 16, 16)] = negv
        return carry

    lax.fori_loop(0, STRIP // 16, init_body, 0)

    def blk(b, carry):
        off = pl.multiple_of(base + b * CH, CH)
        pltpu.sync_copy(lin_hbm.at[pl.ds(off, CH)], lin_b)
        pltpu.sync_copy(key_hbm.at[pl.ds(off, CH)], key_b)
        pltpu.sync_copy(c30_hbm.at[pl.ds(off, CH)], c_b)

        def one(voff):
            l = lin_b[pl.ds(voff, 16)]
            k = key_b[pl.ds(voff, 16)]
            c = c_b[pl.ds(voff, 16)]
            li = l - strip_lo
            m = (li >= 0) & (li < STRIP)
            lic = jnp.where(m, li, 0)
            d = plsc.load_gather(d_tile, [lic])
            need = m & (k < d)
            cnt = plsc.all_reduce_population_count(need)[0]

            @pl.when(cnt > 0)
            def _():
                def step(mask):
                    plsc.store_scatter(d_tile, [lic], k, mask=mask)
                    d2 = plsc.load_gather(d_tile, [lic])
                    won = mask & (k == d2)
                    plsc.store_scatter(c_tile, [lic], c, mask=won)
                    return mask & (k < d2)

                def w_cond(mask):
                    return plsc.all_reduce_population_count(mask)[0] > 0

                lax.while_loop(w_cond, step, step(need))

        def vec(i, c2):
            voff = pl.multiple_of(i * 32, 32)
            one(voff)
            one(voff + 16)
            return c2

        lax.fori_loop(0, CH // 32, vec, 0)
        return carry

    lax.fori_loop(0, CPTS // CH, blk, 0)

    poff = pl.multiple_of(wid * STRIP, STRIP)
    pltpu.sync_copy(d_tile, dpart_hbm.at[pl.ds(poff, STRIP)])
    pltpu.sync_copy(c_tile, cpart_hbm.at[pl.ds(poff, STRIP)])
    plsc.subcore_barrier()

    qo = chunk * QUARTER
    for t in range(QUARTER // TS):
        for j in range(4):
            doff = pl.multiple_of((base_row + j) * STRIP + qo + t * TS, TS)
            pltpu.sync_copy(dpart_hbm.at[pl.ds(doff, TS)],
                            lin_b.at[pl.ds(j * TS, TS)])
            pltpu.sync_copy(cpart_hbm.at[pl.ds(doff, TS)],
                            key_b.at[pl.ds(j * TS, TS)])

        def mv(i, c2, _t=t):
            d0 = lin_b[pl.ds(0 * TS + i * 16, 16)]
            d1 = lin_b[pl.ds(1 * TS + i * 16, 16)]
            d2 = lin_b[pl.ds(2 * TS + i * 16, 16)]
            d3 = lin_b[pl.ds(3 * TS + i * 16, 16)]
            c0 = key_b[pl.ds(0 * TS + i * 16, 16)]
            c1 = key_b[pl.ds(1 * TS + i * 16, 16)]
            c2_ = key_b[pl.ds(2 * TS + i * 16, 16)]
            c3 = key_b[pl.ds(3 * TS + i * 16, 16)]
            d01 = jnp.minimum(d0, d1)
            c01 = jnp.where(d0 <= d1, c0, c1)
            d23 = jnp.minimum(d2, d3)
            c23 = jnp.where(d2 <= d3, c2_, c3)
            c_b[pl.ds(i * 16, 16)] = jnp.where(d01 <= d23, c01, c23)
            return c2

        lax.fori_loop(0, TS // 16, mv, 0)
        ooff = pl.multiple_of(strip_lo + qo + t * TS, TS)
        pltpu.sync_copy(c_b.at[pl.ds(0, TS)], out_hbm.at[pl.ds(ooff, TS)])


def _unpack_body(c_r, r_r, g_r, b_r):
    c = c_r[...]
    cov = c >= 0

    def ch(shift):
        q = (c >> shift) & 1023
        return jnp.where(cov, (q.astype(jnp.float32) + 0.5) * (1.0 / 1024.0),
                         0.0)

    r_r[...] = ch(20)
    g_r[...] = ch(10)
    b_r[...] = ch(0)


def kernel(points, colors):
    xs = jnp.pad(points[:, 0], (0, PAD)).reshape(PROWS, 128)
    ys = jnp.pad(points[:, 1], (0, PAD)).reshape(PROWS, 128)
    zs = jnp.pad(points[:, 2], (0, PAD),
                 constant_values=-3.0).reshape(PROWS, 128)
    rs = jnp.pad(colors[:, 0], (0, PAD)).reshape(PROWS, 128)
    gs = jnp.pad(colors[:, 1], (0, PAD)).reshape(PROWS, 128)
    bs = jnp.pad(colors[:, 2], (0, PAD)).reshape(PROWS, 128)
    lin, key, c30 = pl.pallas_call(
        _prep_body,
        grid=(16,),
        in_specs=[pl.BlockSpec((1024, 128), lambda i: (i, 0))] * 6,
        out_specs=[pl.BlockSpec((1024, 128), lambda i: (i, 0))] * 3,
        out_shape=[jax.ShapeDtypeStruct((PROWS, 128), jnp.int32)] * 3,
    )(xs, ys, zs, rs, gs, bs)
    c30m, _, _ = _sc_scatter(lin.reshape(PN), key.reshape(PN),
                             c30.reshape(PN))
    cr, cg, cb = pl.pallas_call(
        _unpack_body,
        out_shape=[jax.ShapeDtypeStruct((OROWS, 128), jnp.float32)] * 3,
    )(c30m.reshape(OROWS, 128))
    return jnp.stack([cr, cg, cb], axis=-1).reshape(H, W, 3)


# speedup vs baseline: 15.6853x; 1.0004x over previous
import functools

import jax
import jax.numpy as jnp
from jax import lax
from jax.experimental import pallas as pl
from jax.experimental.pallas import tpu as pltpu
from jax.experimental.pallas import tpu_sc as plsc

H, W = 480, 640
P = H * W
FX, FY, CX, CY = 500.0, 500.0, 320.0, 240.0
NPTS = 2_000_000
PN = 2 ** 21
PAD = PN - NPTS
SENT = 2147483647

NSTRIP = 8
STRIP = P // NSTRIP
NCHUNK = 4
CPTS = PN // NCHUNK
CH = 16384
TS = 960
QUARTER = STRIP // NCHUNK

PROWS = PN // 128
OROWS = P // 128


def _prep_body(x_r, y_r, z_r, r_r, g_r, b_r, lin_r, key_r, c30_r):
    z = z_r[...] + 3.0
    u = jnp.round(FX * x_r[...] / z + CX).astype(jnp.int32)
    v = jnp.round(FY * y_r[...] / z + CY).astype(jnp.int32)
    valid = (z > 1e-6) & (u >= 0) & (u < W) & (v >= 0) & (v < H)
    lin_r[...] = jnp.where(valid, v * W + u, SENT)
    zb = lax.bitcast_convert_type(z, jnp.int32)
    key_r[...] = jnp.where(valid, zb, SENT)

    def q(c_ref):
        return jnp.clip((c_ref[...] * 1024.0).astype(jnp.int32), 0, 1023)

    c30_r[...] = (q(r_r) << 20) | (q(g_r) << 10) | q(b_r)


_sc_mesh = plsc.VectorSubcoreMesh(core_axis_name="c", subcore_axis_name="s")


@functools.partial(
    pl.kernel,
    out_type=(
        jax.ShapeDtypeStruct((P,), jnp.int32),
        jax.ShapeDtypeStruct((32 * STRIP,), jnp.int32),
        jax.ShapeDtypeStruct((32 * STRIP,), jnp.int32),
    ),
    mesh=_sc_mesh,
    compiler_params=pltpu.CompilerParams(needs_layout_passes=False),
    scratch_types=[
        pltpu.VMEM((STRIP,), jnp.int32),
        pltpu.VMEM((STRIP,), jnp.int32),
        pltpu.VMEM((CH,), jnp.int32),
        pltpu.VMEM((CH,), jnp.int32),
        pltpu.VMEM((CH,), jnp.int32),
    ],
)
def _sc_scatter(lin_hbm, key_hbm, c30_hbm, out_hbm, dpart_hbm, cpart_hbm,
                d_tile, c_tile, lin_b, key_b, c_b):
    cid = lax.axis_index("c")
    sid = lax.axis_index("s")
    strip = 4 * cid + sid // 4
    chunk = sid % 4
    strip_lo = strip * STRIP
    base = chunk * CPTS
    wid = cid * 16 + sid
    base_row = cid * 16 + (sid // 4) * 4

    maxv = jnp.full((16,), SENT, jnp.int32)
    negv = jnp.full((16,), -1, jnp.int32)

    def init_body(i, carry):
        d_tile[pl.ds(i * 16, 16)] = maxv
        c_tile[pl.ds(i * 16, 16)] = negv
        return carry

    lax.fori_loop(0, STRIP // 16, init_body, 0)

    def blk(b, carry):
        off = pl.multiple_of(base + b * CH, CH)
        pltpu.sync_copy(lin_hbm.at[pl.ds(off, CH)], lin_b)
        pltpu.sync_copy(key_hbm.at[pl.ds(off, CH)], key_b)
        pltpu.sync_copy(c30_hbm.at[pl.ds(off, CH)], c_b)

        def one(voff):
            l = lin_b[pl.ds(voff, 16)]
            k = key_b[pl.ds(voff, 16)]
            li = l - strip_lo
            m = (li >= 0) & (li < STRIP)
            lic = jnp.where(m, li, 0)
            d = plsc.load_gather(d_tile, [lic])
            need = m & (k < d)
            cnt = plsc.all_reduce_population_count(need)[0]

            @pl.when(cnt > 0)
            def _():
                c = c_b[pl.ds(voff, 16)]

                def step(mask):
                    plsc.store_scatter(d_tile, [lic], k, mask=mask)
                    d2 = plsc.load_gather(d_tile, [lic])
                    won = mask & (k == d2)
                    plsc.store_scatter(c_tile, [lic], c, mask=won)
                    return mask & (k < d2)

                def w_cond(mask):
                    return plsc.all_reduce_population_count(mask)[0] > 0

                lax.while_loop(w_cond, step, step(need))

        def vec(i, c2):
            voff = pl.multiple_of(i * 32, 32)
            one(voff)
            one(voff + 16)
            return c2

        lax.fori_loop(0, CH // 32, vec, 0)
        return carry

    lax.fori_loop(0, CPTS // CH, blk, 0)

    poff = pl.multiple_of(wid * STRIP, STRIP)
    pltpu.sync_copy(d_tile, dpart_hbm.at[pl.ds(poff, STRIP)])
    pltpu.sync_copy(c_tile, cpart_hbm.at[pl.ds(poff, STRIP)])
    plsc.subcore_barrier()

    qo = chunk * QUARTER
    for t in range(QUARTER // TS):
        for j in range(4):
            doff = pl.multiple_of((base_row + j) * STRIP + qo + t * TS, TS)
            pltpu.sync_copy(dpart_hbm.at[pl.ds(doff, TS)],
                            lin_b.at[pl.ds(j * TS, TS)])
            pltpu.sync_copy(cpart_hbm.at[pl.ds(doff, TS)],
                            key_b.at[pl.ds(j * TS, TS)])

        def mv(i, c2, _t=t):
            d0 = lin_b[pl.ds(0 * TS + i * 16, 16)]
            d1 = lin_b[pl.ds(1 * TS + i * 16, 16)]
            d2 = lin_b[pl.ds(2 * TS + i * 16, 16)]
            d3 = lin_b[pl.ds(3 * TS + i * 16, 16)]
            c0 = key_b[pl.ds(0 * TS + i * 16, 16)]
            c1 = key_b[pl.ds(1 * TS + i * 16, 16)]
            c2_ = key_b[pl.ds(2 * TS + i * 16, 16)]
            c3 = key_b[pl.ds(3 * TS + i * 16, 16)]
            d01 = jnp.minimum(d0, d1)
            c01 = jnp.where(d0 <= d1, c0, c1)
            d23 = jnp.minimum(d2, d3)
            c23 = jnp.where(d2 <= d3, c2_, c3)
            c_b[pl.ds(i * 16, 16)] = jnp.where(d01 <= d23, c01, c23)
            return c2

        lax.fori_loop(0, TS // 16, mv, 0)
        ooff = pl.multiple_of(strip_lo + qo + t * TS, TS)
        pltpu.sync_copy(c_b.at[pl.ds(0, TS)], out_hbm.at[pl.ds(ooff, TS)])


def _unpack_body(c_r, r_r, g_r, b_r):
    c = c_r[...]
    cov = c >= 0

    def ch(shift):
        q = (c >> shift) & 1023
        return jnp.where(cov, (q.astype(jnp.float32) + 0.5) * (1.0 / 1024.0),
                         0.0)

    r_r[...] = ch(20)
    g_r[...] = ch(10)
    b_r[...] = ch(0)


def kernel(points, colors):
    xs = jnp.pad(points[:, 0], (0, PAD)).reshape(PROWS, 128)
    ys = jnp.pad(points[:, 1], (0, PAD)).reshape(PROWS, 128)
    zs = jnp.pad(points[:, 2], (0, PAD),
                 constant_values=-3.0).reshape(PROWS, 128)
    rs = jnp.pad(colors[:, 0], (0, PAD)).reshape(PROWS, 128)
    gs = jnp.pad(colors[:, 1], (0, PAD)).reshape(PROWS, 128)
    bs = jnp.pad(colors[:, 2], (0, PAD)).reshape(PROWS, 128)
    lin, key, c30 = pl.pallas_call(
        _prep_body,
        grid=(16,),
        in_specs=[pl.BlockSpec((1024, 128), lambda i: (i, 0))] * 6,
        out_specs=[pl.BlockSpec((1024, 128), lambda i: (i, 0))] * 3,
        out_shape=[jax.ShapeDtypeStruct((PROWS, 128), jnp.int32)] * 3,
    )(xs, ys, zs, rs, gs, bs)
    c30m, _, _ = _sc_scatter(lin.reshape(PN), key.reshape(PN),
                             c30.reshape(PN))
    cr, cg, cb = pl.pallas_call(
        _unpack_body,
        out_shape=[jax.ShapeDtypeStruct((OROWS, 128), jnp.float32)] * 3,
    )(c30m.reshape(OROWS, 128))
    return jnp.stack([cr, cg, cb], axis=-1).reshape(H, W, 3)
